# baseline (speedup 1.0000x reference)
import dataclasses

import jax
import jax.numpy as jnp
from jax import lax
from jax.experimental import pallas as pl
from jax.experimental.pallas import tpu as pltpu
from jax.experimental.pallas import tpu_sc as plsc

N = 10000
E = 160000
D = 256
LANES = 16
NTILES = 32
NPT = 320
N_PAD = NTILES * NPT
CHUNK = 2000
NCHUNKS = E // CHUNK
GROUP = 32

_MESH = plsc.VectorSubcoreMesh(core_axis_name="c", subcore_axis_name="s")

_SC_PARAMS = pltpu.CompilerParams()
if "needs_layout_passes" in pltpu.CompilerParams.__dataclass_fields__:
    _SC_PARAMS = dataclasses.replace(_SC_PARAMS, needs_layout_passes=False)


def _tile_range():
    wid = lax.axis_index("c") * 16 + lax.axis_index("s")
    lo = wid * NPT
    return lo, lo + NPT


def _deg_body(col_hbm, ew_hbm, deg_hbm,
              colbuf0, colbuf1, ewbuf0, ewbuf1, deg16, sem0, sem1):
    lo, hi = _tile_range()
    zeros = jnp.zeros((LANES,), jnp.float32)
    bufs = ((colbuf0, ewbuf0, sem0), (colbuf1, ewbuf1, sem1))

    @pl.loop(0, NPT)
    def _(i):
        deg16[i, :] = zeros

    lane = lax.iota(jnp.int32, LANES)

    def _issue(ci, b):
        base = ci * CHUNK
        cb, eb, sm = bufs[b]
        pltpu.async_copy(col_hbm.at[pl.ds(base, CHUNK)], cb, sm.at[0])
        pltpu.async_copy(ew_hbm.at[pl.ds(base, CHUNK)], eb, sm.at[1])

    def _wait(ci, b):
        base = ci * CHUNK
        cb, eb, sm = bufs[b]
        pltpu.make_async_copy(col_hbm.at[pl.ds(base, CHUNK)], cb,
                              sm.at[0]).wait()
        pltpu.make_async_copy(ew_hbm.at[pl.ds(base, CHUNK)], eb,
                              sm.at[1]).wait()

    def _process(b):
        cb, eb, _ = bufs[b]

        @pl.loop(0, CHUNK, step=LANES, unroll=4)
        def _(i):
            col16 = cb[pl.ds(i, LANES)]
            ew16 = eb[pl.ds(i, LANES)]
            msk = (col16 >= lo) & (col16 < hi)
            loc = jnp.where(msk, col16 - lo, 0)
            plsc.addupdate_scatter(deg16, [loc, lane], ew16, mask=msk)

    _issue(0, 0)

    @pl.loop(0, NCHUNKS, step=2)
    def _(ci):
        _wait(ci, 0)
        _issue(ci + 1, 1)
        _process(0)
        _wait(ci + 1, 1)

        @pl.when(ci + 2 < NCHUNKS)
        def _():
            _issue(ci + 2, 0)

        _process(1)

    pltpu.sync_copy(deg16, deg_hbm.at[pl.ds(lo, NPT)])


@jax.jit
def _sc_deg(col, ew):
    kfn = pl.kernel(
        _deg_body,
        out_type=jax.ShapeDtypeStruct((N_PAD, LANES), jnp.float32),
        mesh=_MESH,
        compiler_params=_SC_PARAMS,
        scratch_types=[
            pltpu.VMEM((CHUNK,), jnp.int32),
            pltpu.VMEM((CHUNK,), jnp.int32),
            pltpu.VMEM((CHUNK,), jnp.float32),
            pltpu.VMEM((CHUNK,), jnp.float32),
            pltpu.VMEM((NPT, LANES), jnp.float32),
            pltpu.SemaphoreType.DMA((2,)),
            pltpu.SemaphoreType.DMA((2,)),
        ],
    )
    return kfn(col, ew)


_BM = 400
_GRID_M = N // _BM


def _mm_body(x_ref, w_ref, deg_ref, h_ref, dis_ref):
    h_ref[...] = lax.dot_general(
        x_ref[...], w_ref[...], (((1,), (0,)), ((), ())),
        precision=lax.Precision.HIGHEST)

    @pl.when(pl.program_id(0) == 0)
    def _():
        deg = jnp.sum(deg_ref[...], axis=1) + 1.0
        dis_ref[...] = lax.rsqrt(deg)


@jax.jit
def _tc_mm(x, w, deg16):
    return pl.pallas_call(
        _mm_body,
        grid=(_GRID_M,),
        in_specs=[
            pl.BlockSpec((_BM, D), lambda i: (i, 0)),
            pl.BlockSpec((D, D), lambda i: (0, 0)),
            pl.BlockSpec((N_PAD, LANES), lambda i: (0, 0)),
        ],
        out_specs=[
            pl.BlockSpec((_BM, D), lambda i: (i, 0)),
            pl.BlockSpec((N_PAD,), lambda i: (0,)),
        ],
        out_shape=[
            jax.ShapeDtypeStruct((N, D), jnp.float32),
            jax.ShapeDtypeStruct((N_PAD,), jnp.float32),
        ],
    )(x, w, deg16)


def _main_body(row_hbm, col_hbm, ew_hbm, dis_hbm, h_hbm, acc_hbm,
               disbuf, colbuf0, colbuf1, rowbuf0, rowbuf1, ewbuf0, ewbuf1,
               locbuf, rowselbuf, ewselbuf, normbuf, rows0, rows1, accbuf,
               sem0, sem1, gsem0, gsem1):
    lo, hi = _tile_range()
    zeros = jnp.zeros((LANES,), jnp.float32)
    bufs = ((colbuf0, rowbuf0, ewbuf0, sem0), (colbuf1, rowbuf1, ewbuf1, sem1))
    rbufs = ((rows0, gsem0), (rows1, gsem1))

    def _issue(ci, b):
        base = ci * CHUNK
        cb, rb, eb, sm = bufs[b]
        pltpu.async_copy(col_hbm.at[pl.ds(base, CHUNK)], cb, sm.at[0])
        pltpu.async_copy(row_hbm.at[pl.ds(base, CHUNK)], rb, sm.at[1])
        pltpu.async_copy(ew_hbm.at[pl.ds(base, CHUNK)], eb, sm.at[2])

    def _wait(ci, b):
        base = ci * CHUNK
        cb, rb, eb, sm = bufs[b]
        pltpu.make_async_copy(col_hbm.at[pl.ds(base, CHUNK)], cb,
                              sm.at[0]).wait()
        pltpu.make_async_copy(row_hbm.at[pl.ds(base, CHUNK)], rb,
                              sm.at[1]).wait()
        pltpu.make_async_copy(ew_hbm.at[pl.ds(base, CHUNK)], eb,
                              sm.at[2]).wait()

    def _gissue(g, b):
        rv, gsm = rbufs[b]
        pltpu.async_copy(h_hbm.at[rowselbuf.at[pl.ds(g * GROUP, GROUP)]],
                         rv, gsm)

    def _gwait(g, b):
        rv, gsm = rbufs[b]
        pltpu.make_async_copy(
            h_hbm.at[rowselbuf.at[pl.ds(g * GROUP, GROUP)]], rv, gsm).wait()

    def _process(b):
        cb, rb, eb, _ = bufs[b]

        def compact(i, s_splat):
            col16 = cb[pl.ds(i * LANES, LANES)]
            msk = (col16 >= lo) & (col16 < hi)
            loc16 = jnp.where(msk, col16 - lo, 0)
            row16 = rb[pl.ds(i * LANES, LANES)]
            ew16 = eb[pl.ds(i * LANES, LANES)]
            s = s_splat[0]
            plsc.store_compressed(locbuf.at[pl.ds(s, LANES)], loc16, mask=msk)
            plsc.store_compressed(rowselbuf.at[pl.ds(s, LANES)], row16,
                                  mask=msk)
            plsc.store_compressed(ewselbuf.at[pl.ds(s, LANES)], ew16, mask=msk)
            return s_splat + plsc.all_reduce_population_count(msk)

        s_tot = lax.fori_loop(0, CHUNK // LANES, compact,
                              jnp.zeros((LANES,), jnp.int32), unroll=2)[0]
        zpad = jnp.zeros((LANES,), jnp.int32)
        rowselbuf[pl.ds(s_tot, LANES)] = zpad
        rowselbuf[pl.ds(s_tot + LANES, LANES)] = zpad

        ngroups = (s_tot + (GROUP - 1)) >> 5

        def normloop(j, _):
            r16 = rowselbuf[pl.ds(j * LANES, LANES)]
            disg = plsc.load_gather(disbuf, [r16])
            normbuf[pl.ds(j * LANES, LANES)] = (
                ewselbuf[pl.ds(j * LANES, LANES)] * disg)
            return 0

        lax.fori_loop(0, (s_tot + (LANES - 1)) >> 4, normloop, 0)

        def _accum(g, b2):
            rv, _ = rbufs[b2]
            e_hi = jnp.minimum(s_tot - g * GROUP, GROUP)

            def ebody(el, _):
                e = g * GROUP + el
                loc_e = locbuf[pl.ds(e, LANES)][0]
                nsp = plsc.load_gather(
                    normbuf, [jnp.full((LANES,), e, jnp.int32)])
                vals = [rv[el, pl.ds(d * LANES, LANES)]
                        for d in range(D // LANES)]
                for d in range(D // LANES):
                    sl = pl.ds(d * LANES, LANES)
                    plsc.addupdate(accbuf.at[loc_e, sl], vals[d] * nsp)
                return 0

            lax.fori_loop(0, e_hi, ebody, 0)

        @pl.when(ngroups > 0)
        def _():
            _gissue(0, 0)

        def gpair(p, _):
            g = 2 * p
            _gwait(g, 0)

            @pl.when(g + 1 < ngroups)
            def _():
                _gissue(g + 1, 1)

            _accum(g, 0)

            @pl.when(g + 1 < ngroups)
            def _():
                _gwait(g + 1, 1)

                @pl.when(g + 2 < ngroups)
                def _():
                    _gissue(g + 2, 0)

                _accum(g + 1, 1)

            return 0

        lax.fori_loop(0, (ngroups + 1) >> 1, gpair, 0)

    _issue(0, 0)
    pltpu.sync_copy(dis_hbm, disbuf)

    @pl.loop(0, NPT)
    def _(i):
        for d in range(D // LANES):
            accbuf[i, pl.ds(d * LANES, LANES)] = zeros

    @pl.loop(0, NCHUNKS, step=2)
    def _(ci):
        _wait(ci, 0)
        _issue(ci + 1, 1)
        _process(0)
        _wait(ci + 1, 1)

        @pl.when(ci + 2 < NCHUNKS)
        def _():
            _issue(ci + 2, 0)

        _process(1)

    pltpu.sync_copy(accbuf, acc_hbm.at[pl.ds(lo, NPT)])


@jax.jit
def _sc_main(row, col, ew, dis, h):
    kfn = pl.kernel(
        _main_body,
        out_type=jax.ShapeDtypeStruct((N_PAD, D), jnp.float32),
        mesh=_MESH,
        compiler_params=_SC_PARAMS,
        scratch_types=[
            pltpu.VMEM((N_PAD,), jnp.float32),
            pltpu.VMEM((CHUNK,), jnp.int32),
            pltpu.VMEM((CHUNK,), jnp.int32),
            pltpu.VMEM((CHUNK,), jnp.int32),
            pltpu.VMEM((CHUNK,), jnp.int32),
            pltpu.VMEM((CHUNK,), jnp.float32),
            pltpu.VMEM((CHUNK,), jnp.float32),
            pltpu.VMEM((CHUNK + 2 * LANES,), jnp.int32),
            pltpu.VMEM((CHUNK + 2 * LANES,), jnp.int32),
            pltpu.VMEM((CHUNK + 2 * LANES,), jnp.float32),
            pltpu.VMEM((CHUNK + 2 * LANES,), jnp.float32),
            pltpu.VMEM((GROUP, D), jnp.float32),
            pltpu.VMEM((GROUP, D), jnp.float32),
            pltpu.VMEM((NPT, D), jnp.float32),
            pltpu.SemaphoreType.DMA((3,)),
            pltpu.SemaphoreType.DMA((3,)),
            pltpu.SemaphoreType.DMA,
            pltpu.SemaphoreType.DMA,
        ],
    )
    return kfn(row, col, ew, dis, h)


def _combine_body(acc_ref, h_ref, deg_ref, b_ref, out_ref):
    dis = lax.rsqrt(jnp.sum(deg_ref[...], axis=1, keepdims=True) + 1.0)
    pre = dis * acc_ref[...] + (dis * dis) * h_ref[...] + b_ref[...][None, :]
    out_ref[...] = jnp.maximum(pre, 0.0)


@jax.jit
def _tc_combine(acc, h, deg16, b):
    return pl.pallas_call(
        _combine_body,
        grid=(_GRID_M,),
        in_specs=[
            pl.BlockSpec((_BM, D), lambda i: (i, 0)),
            pl.BlockSpec((_BM, D), lambda i: (i, 0)),
            pl.BlockSpec((_BM, LANES), lambda i: (i, 0)),
            pl.BlockSpec((D,), lambda i: (0,)),
        ],
        out_specs=pl.BlockSpec((_BM, D), lambda i: (i, 0)),
        out_shape=jax.ShapeDtypeStruct((N, D), jnp.float32),
    )(acc, h, deg16, b)


def kernel(x, edge_index, edge_weight, W, b):
    row = edge_index[0]
    col = edge_index[1]
    deg16 = _sc_deg(col, edge_weight)
    h, dis = _tc_mm(x, W, deg16)
    acc = _sc_main(row, col, edge_weight, dis, h)
    return _tc_combine(acc, h, deg16, b)


# speedup vs baseline: 1.6984x; 1.6984x over previous
import dataclasses

import jax
import jax.numpy as jnp
from jax import lax
from jax.experimental import pallas as pl
from jax.experimental.pallas import tpu as pltpu
from jax.experimental.pallas import tpu_sc as plsc

N = 10000
E = 160000
D = 256
LANES = 16
NTILES = 32
NPT = 320
N_PAD = NTILES * NPT
CHUNK = 2000
NCHUNKS = E // CHUNK
GROUP = 16

_MESH = plsc.VectorSubcoreMesh(core_axis_name="c", subcore_axis_name="s")

_SC_PARAMS = pltpu.CompilerParams()
if "needs_layout_passes" in pltpu.CompilerParams.__dataclass_fields__:
    _SC_PARAMS = dataclasses.replace(_SC_PARAMS, needs_layout_passes=False)


def _tile_range():
    wid = lax.axis_index("c") * 16 + lax.axis_index("s")
    lo = wid * NPT
    return lo, lo + NPT


def _deg_body(col_hbm, ew_hbm, deg_hbm,
              colbuf0, colbuf1, ewbuf0, ewbuf1, deg16, sem0, sem1):
    lo, hi = _tile_range()
    zeros = jnp.zeros((LANES,), jnp.float32)
    bufs = ((colbuf0, ewbuf0, sem0), (colbuf1, ewbuf1, sem1))

    @pl.loop(0, NPT)
    def _(i):
        deg16[i, :] = zeros

    lane = lax.iota(jnp.int32, LANES)

    def _issue(ci, b):
        base = ci * CHUNK
        cb, eb, sm = bufs[b]
        pltpu.async_copy(col_hbm.at[pl.ds(base, CHUNK)], cb, sm.at[0])
        pltpu.async_copy(ew_hbm.at[pl.ds(base, CHUNK)], eb, sm.at[1])

    def _wait(ci, b):
        base = ci * CHUNK
        cb, eb, sm = bufs[b]
        pltpu.make_async_copy(col_hbm.at[pl.ds(base, CHUNK)], cb,
                              sm.at[0]).wait()
        pltpu.make_async_copy(ew_hbm.at[pl.ds(base, CHUNK)], eb,
                              sm.at[1]).wait()

    def _process(b):
        cb, eb, _ = bufs[b]

        @pl.loop(0, CHUNK, step=LANES, unroll=4)
        def _(i):
            col16 = cb[pl.ds(i, LANES)]
            ew16 = eb[pl.ds(i, LANES)]
            msk = (col16 >= lo) & (col16 < hi)
            loc = jnp.where(msk, col16 - lo, 0)
            plsc.addupdate_scatter(deg16, [loc, lane], ew16, mask=msk)

    _issue(0, 0)

    @pl.loop(0, NCHUNKS, step=2)
    def _(ci):
        _wait(ci, 0)
        _issue(ci + 1, 1)
        _process(0)
        _wait(ci + 1, 1)

        @pl.when(ci + 2 < NCHUNKS)
        def _():
            _issue(ci + 2, 0)

        _process(1)

    pltpu.sync_copy(deg16, deg_hbm.at[pl.ds(lo, NPT)])


@jax.jit
def _sc_deg(col, ew):
    kfn = pl.kernel(
        _deg_body,
        out_type=jax.ShapeDtypeStruct((N_PAD, LANES), jnp.float32),
        mesh=_MESH,
        compiler_params=_SC_PARAMS,
        scratch_types=[
            pltpu.VMEM((CHUNK,), jnp.int32),
            pltpu.VMEM((CHUNK,), jnp.int32),
            pltpu.VMEM((CHUNK,), jnp.float32),
            pltpu.VMEM((CHUNK,), jnp.float32),
            pltpu.VMEM((NPT, LANES), jnp.float32),
            pltpu.SemaphoreType.DMA((2,)),
            pltpu.SemaphoreType.DMA((2,)),
        ],
    )
    return kfn(col, ew)


_BM = 400
_GRID_M = N // _BM


def _mm_body(x_ref, w_ref, deg_ref, h_ref, dis_ref):
    h_ref[...] = lax.dot_general(
        x_ref[...], w_ref[...], (((1,), (0,)), ((), ())),
        precision=lax.Precision.HIGHEST)

    @pl.when(pl.program_id(0) == 0)
    def _():
        deg = jnp.sum(deg_ref[...], axis=1) + 1.0
        dis_ref[...] = lax.rsqrt(deg)


@jax.jit
def _tc_mm(x, w, deg16):
    return pl.pallas_call(
        _mm_body,
        grid=(_GRID_M,),
        in_specs=[
            pl.BlockSpec((_BM, D), lambda i: (i, 0)),
            pl.BlockSpec((D, D), lambda i: (0, 0)),
            pl.BlockSpec((N_PAD, LANES), lambda i: (0, 0)),
        ],
        out_specs=[
            pl.BlockSpec((_BM, D), lambda i: (i, 0)),
            pl.BlockSpec((N_PAD,), lambda i: (0,)),
        ],
        out_shape=[
            jax.ShapeDtypeStruct((N, D), jnp.float32),
            jax.ShapeDtypeStruct((N_PAD,), jnp.float32),
        ],
    )(x, w, deg16)


def _main_body(row_hbm, col_hbm, ew_hbm, dis_hbm, h_hbm, acc_hbm,
               disbuf, colbuf0, colbuf1, rowbuf0, rowbuf1, ewbuf0, ewbuf1,
               locbuf, rowselbuf, ewselbuf, normbuf, rows0, rows1, accbuf,
               sem0, sem1, gsem0, gsem1):
    lo, hi = _tile_range()
    zeros = jnp.zeros((LANES,), jnp.float32)
    bufs = ((colbuf0, rowbuf0, ewbuf0, sem0), (colbuf1, rowbuf1, ewbuf1, sem1))
    rbufs = ((rows0, gsem0), (rows1, gsem1))

    def _issue(ci, b):
        base = ci * CHUNK
        cb, rb, eb, sm = bufs[b]
        pltpu.async_copy(col_hbm.at[pl.ds(base, CHUNK)], cb, sm.at[0])
        pltpu.async_copy(row_hbm.at[pl.ds(base, CHUNK)], rb, sm.at[1])
        pltpu.async_copy(ew_hbm.at[pl.ds(base, CHUNK)], eb, sm.at[2])

    def _wait(ci, b):
        base = ci * CHUNK
        cb, rb, eb, sm = bufs[b]
        pltpu.make_async_copy(col_hbm.at[pl.ds(base, CHUNK)], cb,
                              sm.at[0]).wait()
        pltpu.make_async_copy(row_hbm.at[pl.ds(base, CHUNK)], rb,
                              sm.at[1]).wait()
        pltpu.make_async_copy(ew_hbm.at[pl.ds(base, CHUNK)], eb,
                              sm.at[2]).wait()

    def _gissue(g, b):
        rv, gsm = rbufs[b]
        rn = rowselbuf[pl.ds(g * GROUP, GROUP)]
        pltpu.async_copy(h_hbm.at[rn], rv, gsm)

    def _gwait(g, b):
        rv, gsm = rbufs[b]
        rn = rowselbuf[pl.ds(g * GROUP, GROUP)]
        pltpu.make_async_copy(h_hbm.at[rn], rv, gsm).wait()

    def _process(b):
        cb, rb, eb, _ = bufs[b]

        def compact(i, s_splat):
            col16 = cb[pl.ds(i * LANES, LANES)]
            msk = (col16 >= lo) & (col16 < hi)
            loc16 = jnp.where(msk, col16 - lo, 0)
            row16 = rb[pl.ds(i * LANES, LANES)]
            ew16 = eb[pl.ds(i * LANES, LANES)]
            s = s_splat[0]
            plsc.store_compressed(locbuf.at[pl.ds(s, LANES)], loc16, mask=msk)
            plsc.store_compressed(rowselbuf.at[pl.ds(s, LANES)], row16,
                                  mask=msk)
            plsc.store_compressed(ewselbuf.at[pl.ds(s, LANES)], ew16, mask=msk)
            return s_splat + plsc.all_reduce_population_count(msk)

        s_tot = lax.fori_loop(0, CHUNK // LANES, compact,
                              jnp.zeros((LANES,), jnp.int32), unroll=2)[0]
        zpad = jnp.zeros((LANES,), jnp.int32)
        rowselbuf[pl.ds(s_tot, LANES)] = zpad
        rowselbuf[pl.ds(s_tot + LANES, LANES)] = zpad

        ngroups = (s_tot + (GROUP - 1)) >> 4

        def normloop(j, _):
            r16 = rowselbuf[pl.ds(j * LANES, LANES)]
            disg = plsc.load_gather(disbuf, [r16])
            normbuf[pl.ds(j * LANES, LANES)] = (
                ewselbuf[pl.ds(j * LANES, LANES)] * disg)
            return 0

        lax.fori_loop(0, (s_tot + (LANES - 1)) >> 4, normloop, 0)

        def _accum(g, b2):
            rv, _ = rbufs[b2]
            e_hi = jnp.minimum(s_tot - g * GROUP, GROUP)

            def ebody(el, _):
                e = g * GROUP + el
                loc_e = locbuf[pl.ds(e, LANES)][0]
                nsp = plsc.load_gather(
                    normbuf, [jnp.full((LANES,), e, jnp.int32)])
                for d in range(D // LANES):
                    sl = pl.ds(d * LANES, LANES)
                    plsc.addupdate(accbuf.at[loc_e, sl], rv[el, sl] * nsp)
                return 0

            lax.fori_loop(0, e_hi, ebody, 0)

        @pl.when(ngroups > 0)
        def _():
            _gissue(0, 0)

        def gpair(p, _):
            g = 2 * p
            _gwait(g, 0)

            @pl.when(g + 1 < ngroups)
            def _():
                _gissue(g + 1, 1)

            _accum(g, 0)

            @pl.when(g + 1 < ngroups)
            def _():
                _gwait(g + 1, 1)

                @pl.when(g + 2 < ngroups)
                def _():
                    _gissue(g + 2, 0)

                _accum(g + 1, 1)

            return 0

        lax.fori_loop(0, (ngroups + 1) >> 1, gpair, 0)

    _issue(0, 0)
    pltpu.sync_copy(dis_hbm, disbuf)

    @pl.loop(0, NPT)
    def _(i):
        for d in range(D // LANES):
            accbuf[i, pl.ds(d * LANES, LANES)] = zeros

    @pl.loop(0, NCHUNKS, step=2)
    def _(ci):
        _wait(ci, 0)
        _issue(ci + 1, 1)
        _process(0)
        _wait(ci + 1, 1)

        @pl.when(ci + 2 < NCHUNKS)
        def _():
            _issue(ci + 2, 0)

        _process(1)

    pltpu.sync_copy(accbuf, acc_hbm.at[pl.ds(lo, NPT)])


@jax.jit
def _sc_main(row, col, ew, dis, h):
    kfn = pl.kernel(
        _main_body,
        out_type=jax.ShapeDtypeStruct((N_PAD, D), jnp.float32),
        mesh=_MESH,
        compiler_params=_SC_PARAMS,
        scratch_types=[
            pltpu.VMEM((N_PAD,), jnp.float32),
            pltpu.VMEM((CHUNK,), jnp.int32),
            pltpu.VMEM((CHUNK,), jnp.int32),
            pltpu.VMEM((CHUNK,), jnp.int32),
            pltpu.VMEM((CHUNK,), jnp.int32),
            pltpu.VMEM((CHUNK,), jnp.float32),
            pltpu.VMEM((CHUNK,), jnp.float32),
            pltpu.VMEM((CHUNK + 2 * LANES,), jnp.int32),
            pltpu.VMEM((CHUNK + 2 * LANES,), jnp.int32),
            pltpu.VMEM((CHUNK + 2 * LANES,), jnp.float32),
            pltpu.VMEM((CHUNK + 2 * LANES,), jnp.float32),
            pltpu.VMEM((GROUP, D), jnp.float32),
            pltpu.VMEM((GROUP, D), jnp.float32),
            pltpu.VMEM((NPT, D), jnp.float32),
            pltpu.SemaphoreType.DMA((3,)),
            pltpu.SemaphoreType.DMA((3,)),
            pltpu.SemaphoreType.DMA,
            pltpu.SemaphoreType.DMA,
        ],
    )
    return kfn(row, col, ew, dis, h)


def _combine_body(acc_ref, h_ref, deg_ref, b_ref, out_ref):
    dis = lax.rsqrt(jnp.sum(deg_ref[...], axis=1, keepdims=True) + 1.0)
    pre = dis * acc_ref[...] + (dis * dis) * h_ref[...] + b_ref[...][None, :]
    out_ref[...] = jnp.maximum(pre, 0.0)


@jax.jit
def _tc_combine(acc, h, deg16, b):
    return pl.pallas_call(
        _combine_body,
        grid=(_GRID_M,),
        in_specs=[
            pl.BlockSpec((_BM, D), lambda i: (i, 0)),
            pl.BlockSpec((_BM, D), lambda i: (i, 0)),
            pl.BlockSpec((_BM, LANES), lambda i: (i, 0)),
            pl.BlockSpec((D,), lambda i: (0,)),
        ],
        out_specs=pl.BlockSpec((_BM, D), lambda i: (i, 0)),
        out_shape=jax.ShapeDtypeStruct((N, D), jnp.float32),
    )(acc, h, deg16, b)


def kernel(x, edge_index, edge_weight, W, b):
    row = edge_index[0]
    col = edge_index[1]
    deg16 = _sc_deg(col, edge_weight)
    h, dis = _tc_mm(x, W, deg16)
    acc = _sc_main(row, col, edge_weight, dis, h)
    return _tc_combine(acc, h, deg16, b)


# speedup vs baseline: 1.7123x; 1.0082x over previous
import dataclasses

import jax
import jax.numpy as jnp
from jax import lax
from jax.experimental import pallas as pl
from jax.experimental.pallas import tpu as pltpu
from jax.experimental.pallas import tpu_sc as plsc

N = 10000
E = 160000
D = 256
LANES = 16
NTILES = 32
NPT = 320
N_PAD = NTILES * NPT
CHUNK = 2000
NCHUNKS = E // CHUNK
GROUP = 16

_MESH = plsc.VectorSubcoreMesh(core_axis_name="c", subcore_axis_name="s")

_SC_PARAMS = pltpu.CompilerParams()
if "needs_layout_passes" in pltpu.CompilerParams.__dataclass_fields__:
    _SC_PARAMS = dataclasses.replace(_SC_PARAMS, needs_layout_passes=False)


def _tile_range():
    wid = lax.axis_index("c") * 16 + lax.axis_index("s")
    lo = wid * NPT
    return lo, lo + NPT


def _deg_body(col_hbm, ew_hbm, deg_hbm,
              colbuf0, colbuf1, ewbuf0, ewbuf1, deg16, sem0, sem1):
    lo, hi = _tile_range()
    zeros = jnp.zeros((LANES,), jnp.float32)
    bufs = ((colbuf0, ewbuf0, sem0), (colbuf1, ewbuf1, sem1))

    @pl.loop(0, NPT)
    def _(i):
        deg16[i, :] = zeros

    lane = lax.iota(jnp.int32, LANES)

    def _issue(ci, b):
        base = ci * CHUNK
        cb, eb, sm = bufs[b]
        pltpu.async_copy(col_hbm.at[pl.ds(base, CHUNK)], cb, sm.at[0])
        pltpu.async_copy(ew_hbm.at[pl.ds(base, CHUNK)], eb, sm.at[1])

    def _wait(ci, b):
        base = ci * CHUNK
        cb, eb, sm = bufs[b]
        pltpu.make_async_copy(col_hbm.at[pl.ds(base, CHUNK)], cb,
                              sm.at[0]).wait()
        pltpu.make_async_copy(ew_hbm.at[pl.ds(base, CHUNK)], eb,
                              sm.at[1]).wait()

    def _process(b):
        cb, eb, _ = bufs[b]

        @pl.loop(0, CHUNK, step=LANES, unroll=4)
        def _(i):
            col16 = cb[pl.ds(i, LANES)]
            ew16 = eb[pl.ds(i, LANES)]
            msk = (col16 >= lo) & (col16 < hi)
            loc = jnp.where(msk, col16 - lo, 0)
            plsc.addupdate_scatter(deg16, [loc, lane], ew16, mask=msk)

    _issue(0, 0)

    @pl.loop(0, NCHUNKS, step=2)
    def _(ci):
        _wait(ci, 0)
        _issue(ci + 1, 1)
        _process(0)
        _wait(ci + 1, 1)

        @pl.when(ci + 2 < NCHUNKS)
        def _():
            _issue(ci + 2, 0)

        _process(1)

    pltpu.sync_copy(deg16, deg_hbm.at[pl.ds(lo, NPT)])


@jax.jit
def _sc_deg(col, ew):
    kfn = pl.kernel(
        _deg_body,
        out_type=jax.ShapeDtypeStruct((N_PAD, LANES), jnp.float32),
        mesh=_MESH,
        compiler_params=_SC_PARAMS,
        scratch_types=[
            pltpu.VMEM((CHUNK,), jnp.int32),
            pltpu.VMEM((CHUNK,), jnp.int32),
            pltpu.VMEM((CHUNK,), jnp.float32),
            pltpu.VMEM((CHUNK,), jnp.float32),
            pltpu.VMEM((NPT, LANES), jnp.float32),
            pltpu.SemaphoreType.DMA((2,)),
            pltpu.SemaphoreType.DMA((2,)),
        ],
    )
    return kfn(col, ew)


_BM = 400
_GRID_M = N // _BM


def _mm_body(x_ref, w_ref, deg_ref, h_ref, dis_ref):
    h_ref[...] = lax.dot_general(
        x_ref[...], w_ref[...], (((1,), (0,)), ((), ())),
        precision=lax.Precision.HIGHEST)

    @pl.when(pl.program_id(0) == 0)
    def _():
        deg = jnp.sum(deg_ref[...], axis=1) + 1.0
        dis_ref[...] = lax.rsqrt(deg)


@jax.jit
def _tc_mm(x, w, deg16):
    return pl.pallas_call(
        _mm_body,
        grid=(_GRID_M,),
        in_specs=[
            pl.BlockSpec((_BM, D), lambda i: (i, 0)),
            pl.BlockSpec((D, D), lambda i: (0, 0)),
            pl.BlockSpec((N_PAD, LANES), lambda i: (0, 0)),
        ],
        out_specs=[
            pl.BlockSpec((_BM, D), lambda i: (i, 0)),
            pl.BlockSpec((N_PAD,), lambda i: (0,)),
        ],
        out_shape=[
            jax.ShapeDtypeStruct((N, D), jnp.float32),
            jax.ShapeDtypeStruct((N_PAD,), jnp.float32),
        ],
    )(x, w, deg16)


def _main_body(row_hbm, col_hbm, ew_hbm, dis_hbm, h_hbm, acc_hbm,
               disbuf, colbuf0, colbuf1, rowbuf0, rowbuf1, ewbuf0, ewbuf1,
               locbuf, rowselbuf, ewselbuf, normbuf, rows0, rows1, accbuf,
               sem0, sem1, gsem0, gsem1):
    lo, hi = _tile_range()
    zeros = jnp.zeros((LANES,), jnp.float32)
    bufs = ((colbuf0, rowbuf0, ewbuf0, sem0), (colbuf1, rowbuf1, ewbuf1, sem1))
    rbufs = ((rows0, gsem0), (rows1, gsem1))

    def _issue(ci, b):
        base = ci * CHUNK
        cb, rb, eb, sm = bufs[b]
        pltpu.async_copy(col_hbm.at[pl.ds(base, CHUNK)], cb, sm.at[0])
        pltpu.async_copy(row_hbm.at[pl.ds(base, CHUNK)], rb, sm.at[1])
        pltpu.async_copy(ew_hbm.at[pl.ds(base, CHUNK)], eb, sm.at[2])

    def _wait(ci, b):
        base = ci * CHUNK
        cb, rb, eb, sm = bufs[b]
        pltpu.make_async_copy(col_hbm.at[pl.ds(base, CHUNK)], cb,
                              sm.at[0]).wait()
        pltpu.make_async_copy(row_hbm.at[pl.ds(base, CHUNK)], rb,
                              sm.at[1]).wait()
        pltpu.make_async_copy(ew_hbm.at[pl.ds(base, CHUNK)], eb,
                              sm.at[2]).wait()

    def _gissue(g, b):
        rv, gsm = rbufs[b]
        rn = rowselbuf[pl.ds(g * GROUP, GROUP)]
        pltpu.async_copy(h_hbm.at[rn], rv, gsm)

    def _gwait(g, b):
        rv, gsm = rbufs[b]
        rn = rowselbuf[pl.ds(g * GROUP, GROUP)]
        pltpu.make_async_copy(h_hbm.at[rn], rv, gsm).wait()

    def _process(b):
        cb, rb, eb, _ = bufs[b]

        def compact(i, s_splat):
            col16 = cb[pl.ds(i * LANES, LANES)]
            msk = (col16 >= lo) & (col16 < hi)
            loc16 = jnp.where(msk, col16 - lo, 0)
            row16 = rb[pl.ds(i * LANES, LANES)]
            ew16 = eb[pl.ds(i * LANES, LANES)]
            s = s_splat[0]
            plsc.store_compressed(locbuf.at[pl.ds(s, LANES)], loc16, mask=msk)
            plsc.store_compressed(rowselbuf.at[pl.ds(s, LANES)], row16,
                                  mask=msk)
            plsc.store_compressed(ewselbuf.at[pl.ds(s, LANES)], ew16, mask=msk)
            return s_splat + plsc.all_reduce_population_count(msk)

        s_tot = lax.fori_loop(0, CHUNK // LANES, compact,
                              jnp.zeros((LANES,), jnp.int32), unroll=2)[0]
        zpad = jnp.zeros((LANES,), jnp.int32)
        rowselbuf[pl.ds(s_tot, LANES)] = zpad
        rowselbuf[pl.ds(s_tot + LANES, LANES)] = zpad

        ngroups = (s_tot + (GROUP - 1)) >> 4

        def normloop(j, _):
            r16 = rowselbuf[pl.ds(j * LANES, LANES)]
            disg = plsc.load_gather(disbuf, [r16])
            normbuf[pl.ds(j * LANES, LANES)] = (
                ewselbuf[pl.ds(j * LANES, LANES)] * disg)
            return 0

        lax.fori_loop(0, (s_tot + (LANES - 1)) >> 4, normloop, 0)

        def _accum(g, b2):
            rv, _ = rbufs[b2]
            e_hi = jnp.minimum(s_tot - g * GROUP, GROUP)

            def ebody(el, _):
                e = g * GROUP + el
                loc_e = locbuf[pl.ds(e, LANES)][0]
                nsp = plsc.load_gather(
                    normbuf, [jnp.full((LANES,), e, jnp.int32)])
                ns = D // LANES
                vs = {0: rv[el, pl.ds(0, LANES)], 1: rv[el, pl.ds(LANES, LANES)]}
                for d in range(ns):
                    if d + 2 < ns:
                        vs[d + 2] = rv[el, pl.ds((d + 2) * LANES, LANES)]
                    sl = pl.ds(d * LANES, LANES)
                    plsc.addupdate(accbuf.at[loc_e, sl], vs.pop(d) * nsp)
                return 0

            lax.fori_loop(0, e_hi, ebody, 0)

        @pl.when(ngroups > 0)
        def _():
            _gissue(0, 0)

        def gpair(p, _):
            g = 2 * p
            _gwait(g, 0)

            @pl.when(g + 1 < ngroups)
            def _():
                _gissue(g + 1, 1)

            _accum(g, 0)

            @pl.when(g + 1 < ngroups)
            def _():
                _gwait(g + 1, 1)

                @pl.when(g + 2 < ngroups)
                def _():
                    _gissue(g + 2, 0)

                _accum(g + 1, 1)

            return 0

        lax.fori_loop(0, (ngroups + 1) >> 1, gpair, 0)

    _issue(0, 0)
    pltpu.sync_copy(dis_hbm, disbuf)

    @pl.loop(0, NPT)
    def _(i):
        for d in range(D // LANES):
            accbuf[i, pl.ds(d * LANES, LANES)] = zeros

    @pl.loop(0, NCHUNKS, step=2)
    def _(ci):
        _wait(ci, 0)
        _issue(ci + 1, 1)
        _process(0)
        _wait(ci + 1, 1)

        @pl.when(ci + 2 < NCHUNKS)
        def _():
            _issue(ci + 2, 0)

        _process(1)

    pltpu.sync_copy(accbuf, acc_hbm.at[pl.ds(lo, NPT)])


@jax.jit
def _sc_main(row, col, ew, dis, h):
    kfn = pl.kernel(
        _main_body,
        out_type=jax.ShapeDtypeStruct((N_PAD, D), jnp.float32),
        mesh=_MESH,
        compiler_params=_SC_PARAMS,
        scratch_types=[
            pltpu.VMEM((N_PAD,), jnp.float32),
            pltpu.VMEM((CHUNK,), jnp.int32),
            pltpu.VMEM((CHUNK,), jnp.int32),
            pltpu.VMEM((CHUNK,), jnp.int32),
            pltpu.VMEM((CHUNK,), jnp.int32),
            pltpu.VMEM((CHUNK,), jnp.float32),
            pltpu.VMEM((CHUNK,), jnp.float32),
            pltpu.VMEM((CHUNK + 2 * LANES,), jnp.int32),
            pltpu.VMEM((CHUNK + 2 * LANES,), jnp.int32),
            pltpu.VMEM((CHUNK + 2 * LANES,), jnp.float32),
            pltpu.VMEM((CHUNK + 2 * LANES,), jnp.float32),
            pltpu.VMEM((GROUP, D), jnp.float32),
            pltpu.VMEM((GROUP, D), jnp.float32),
            pltpu.VMEM((NPT, D), jnp.float32),
            pltpu.SemaphoreType.DMA((3,)),
            pltpu.SemaphoreType.DMA((3,)),
            pltpu.SemaphoreType.DMA,
            pltpu.SemaphoreType.DMA,
        ],
    )
    return kfn(row, col, ew, dis, h)


def _combine_body(acc_ref, h_ref, deg_ref, b_ref, out_ref):
    dis = lax.rsqrt(jnp.sum(deg_ref[...], axis=1, keepdims=True) + 1.0)
    pre = dis * acc_ref[...] + (dis * dis) * h_ref[...] + b_ref[...][None, :]
    out_ref[...] = jnp.maximum(pre, 0.0)


@jax.jit
def _tc_combine(acc, h, deg16, b):
    return pl.pallas_call(
        _combine_body,
        grid=(_GRID_M,),
        in_specs=[
            pl.BlockSpec((_BM, D), lambda i: (i, 0)),
            pl.BlockSpec((_BM, D), lambda i: (i, 0)),
            pl.BlockSpec((_BM, LANES), lambda i: (i, 0)),
            pl.BlockSpec((D,), lambda i: (0,)),
        ],
        out_specs=pl.BlockSpec((_BM, D), lambda i: (i, 0)),
        out_shape=jax.ShapeDtypeStruct((N, D), jnp.float32),
    )(acc, h, deg16, b)


def kernel(x, edge_index, edge_weight, W, b):
    row = edge_index[0]
    col = edge_index[1]
    deg16 = _sc_deg(col, edge_weight)
    h, dis = _tc_mm(x, W, deg16)
    acc = _sc_main(row, col, edge_weight, dis, h)
    return _tc_combine(acc, h, deg16, b)
